# baseline (speedup 1.0000x reference)
import functools

import jax
import jax.numpy as jnp
import numpy as np
from jax.experimental import pallas as pl
from jax.experimental.pallas import tpu as pltpu

_N = 5000
_K = 100
_NPAD = 5120
_R = 1024
_C = 1024
_SLOTS = 128
_HIGH = jax.lax.Precision.HIGHEST

_NB = _NPAD // _R
_CR = _C // 128
_STEPS = [(cb, rb) for cb in range(_NB) for rb in range(cb + 1)]
_TABLE = np.asarray(_STEPS, dtype=np.int32).T.copy()


def _supp_kernel(tb_ref, br_ref, bc_ref, o_ref):
    i = pl.program_id(0)
    cb = tb_ref[0, i]
    rb = tb_ref[1, i]

    b = br_ref[...]
    bt = bc_ref[...].T
    x1i, y1i, x2i, y2i = b[:, 1:2], b[:, 2:3], b[:, 3:4], b[:, 4:5]
    x1j, y1j, x2j, y2j = bt[1:2, :], bt[2:3, :], bt[3:4, :], bt[4:5, :]

    iw = jnp.minimum(x2i, x2j) - jnp.maximum(x1i, x1j)
    ih = jnp.minimum(y2i, y2j) - jnp.maximum(y1i, y1j)
    inter = jnp.maximum(iw, 0.0) * jnp.maximum(ih, 0.0)
    ai = (x2i - x1i) * (y2i - y1i)
    aj = (x2j - x1j) * (y2j - y1j)
    union = (ai + aj) - inter
    d = (inter + inter) - union

    @pl.when(rb == cb)
    def _diag():
        gi = jax.lax.broadcasted_iota(jnp.int32, (_R, 1), 0)
        gj = jax.lax.broadcasted_iota(jnp.int32, (1, _C), 1)
        col = jnp.max(jnp.where(gi < gj, d, -1.0), axis=0,
                      keepdims=True).reshape(1, _CR, 128)
        @pl.when(rb == 0)
        def _init():
            o_ref[...] = col
        @pl.when(rb != 0)
        def _acc():
            o_ref[...] = jnp.maximum(o_ref[...], col)

    @pl.when(rb < cb)
    def _off():
        col = jnp.max(d, axis=0, keepdims=True).reshape(1, _CR, 128)
        @pl.when(rb == 0)
        def _init():
            o_ref[...] = col
        @pl.when(rb != 0)
        def _acc():
            o_ref[...] = jnp.maximum(o_ref[...], col)


def _compact_kernel(dmax_ref, data_ref, o_ref):
    dmax = dmax_ref[...].reshape(_NPAD // 128, 128)
    r_i = jax.lax.broadcasted_iota(jnp.int32, (_NPAD // 128, 128), 0)
    l_i = jax.lax.broadcasted_iota(jnp.int32, (_NPAD // 128, 128), 1)
    keep = (dmax <= 0.0) & ((r_i * 128 + l_i) < _N)
    kf = jnp.where(keep, 1.0, 0.0)

    u_r = jax.lax.broadcasted_iota(jnp.int32, (128, 128), 0)
    u_c = jax.lax.broadcasted_iota(jnp.int32, (128, 128), 1)
    upper = jnp.where(u_r <= u_c, 1.0, 0.0)
    incl = jnp.dot(kf, upper)

    l_r = jax.lax.broadcasted_iota(jnp.int32, (40, 40), 0)
    l_c = jax.lax.broadcasted_iota(jnp.int32, (40, 40), 1)
    lower = jnp.where(l_r > l_c, 1.0, 0.0)
    offs = jnp.dot(lower, incl[:, 127:128], precision=_HIGH)

    rank = (incl + offs - kf).astype(jnp.int32)
    slot = jnp.where(keep, rank, jnp.int32(2**30))
    slot_flat = slot.reshape(1, _NPAD)
    p_i = jax.lax.broadcasted_iota(jnp.int32, (_SLOTS, 1), 0)
    onehot = jnp.where(p_i == slot_flat, 1.0, 0.0)
    res = jnp.dot(onehot, data_ref[...], precision=_HIGH)
    o_ref[...] = res[:_K, :5]


@functools.partial(jax.jit, static_argnames=("interpret",))
def kernel(boxes, scores, interpret=False):
    neg, x1c, y1c, x2c, y2c = jax.lax.sort(
        (-scores, boxes[:, 0], boxes[:, 1], boxes[:, 2], boxes[:, 3]),
        num_keys=1, is_stable=True)
    z = jnp.zeros((_N,), jnp.float32)
    data = jnp.stack([-neg, x1c, y1c, x2c, y2c, z, z, z], axis=1)
    data = jnp.pad(data, ((0, _NPAD - _N), (0, 0)))

    dmax = pl.pallas_call(
        _supp_kernel,
        grid_spec=pltpu.PrefetchScalarGridSpec(
            num_scalar_prefetch=1,
            grid=(len(_STEPS),),
            in_specs=[
                pl.BlockSpec((_R, 8), lambda i, tb: (tb[1, i], 0)),
                pl.BlockSpec((_C, 8), lambda i, tb: (tb[0, i], 0)),
            ],
            out_specs=pl.BlockSpec((1, _CR, 128),
                                   lambda i, tb: (tb[0, i], 0, 0)),
        ),
        out_shape=jax.ShapeDtypeStruct((_NB, _CR, 128), jnp.float32),
        interpret=interpret,
    )(jnp.asarray(_TABLE), data, data)

    return pl.pallas_call(
        _compact_kernel,
        out_shape=jax.ShapeDtypeStruct((_K, 5), jnp.float32),
        interpret=interpret,
    )(dmax, data)


# speedup vs baseline: 1.0172x; 1.0172x over previous
import functools

import jax
import jax.numpy as jnp
import numpy as np
from jax.experimental import pallas as pl
from jax.experimental.pallas import tpu as pltpu

_N = 5000
_K = 100
_NPAD = 5120
_R = 1280
_C = 1280
_SLOTS = 128
_HIGH = jax.lax.Precision.HIGHEST

_NB = _NPAD // _R
_CR = _C // 128
_STEPS = [(cb, rb) for cb in (3, 0, 2, 1) for rb in range(cb + 1)]
_TABLE = np.asarray(_STEPS, dtype=np.int32).T.copy()


def _supp_kernel(tb_ref, br_ref, bc_ref, o_ref):
    i = pl.program_id(0)
    cb = tb_ref[0, i]
    rb = tb_ref[1, i]

    b = br_ref[...]
    bt = bc_ref[...].T
    x1i, y1i, x2i, y2i = b[:, 1:2], b[:, 2:3], b[:, 3:4], b[:, 4:5]
    x1j, y1j, x2j, y2j = bt[1:2, :], bt[2:3, :], bt[3:4, :], bt[4:5, :]

    iw = jnp.minimum(x2i, x2j) - jnp.maximum(x1i, x1j)
    ih = jnp.minimum(y2i, y2j) - jnp.maximum(y1i, y1j)
    inter = jnp.maximum(iw, 0.0) * jnp.maximum(ih, 0.0)
    ai = (x2i - x1i) * (y2i - y1i)
    aj = (x2j - x1j) * (y2j - y1j)
    union = (ai + aj) - inter
    d = (inter + inter) - union

    @pl.when(rb == cb)
    def _diag():
        gi = jax.lax.broadcasted_iota(jnp.int32, (_R, 1), 0)
        gj = jax.lax.broadcasted_iota(jnp.int32, (1, _C), 1)
        col = jnp.max(jnp.where(gi < gj, d, -1.0), axis=0,
                      keepdims=True).reshape(1, _CR, 128)
        @pl.when(rb == 0)
        def _init():
            o_ref[...] = col
        @pl.when(rb != 0)
        def _acc():
            o_ref[...] = jnp.maximum(o_ref[...], col)

    @pl.when(rb < cb)
    def _off():
        col = jnp.max(d, axis=0, keepdims=True).reshape(1, _CR, 128)
        @pl.when(rb == 0)
        def _init():
            o_ref[...] = col
        @pl.when(rb != 0)
        def _acc():
            o_ref[...] = jnp.maximum(o_ref[...], col)


def _compact_kernel(dmax_ref, data_ref, o_ref):
    dmax = dmax_ref[...].reshape(_NPAD // 128, 128)
    r_i = jax.lax.broadcasted_iota(jnp.int32, (_NPAD // 128, 128), 0)
    l_i = jax.lax.broadcasted_iota(jnp.int32, (_NPAD // 128, 128), 1)
    keep = (dmax <= 0.0) & ((r_i * 128 + l_i) < _N)
    kf = jnp.where(keep, 1.0, 0.0)

    u_r = jax.lax.broadcasted_iota(jnp.int32, (128, 128), 0)
    u_c = jax.lax.broadcasted_iota(jnp.int32, (128, 128), 1)
    upper = jnp.where(u_r <= u_c, 1.0, 0.0)
    incl = jnp.dot(kf, upper)

    l_r = jax.lax.broadcasted_iota(jnp.int32, (40, 40), 0)
    l_c = jax.lax.broadcasted_iota(jnp.int32, (40, 40), 1)
    lower = jnp.where(l_r > l_c, 1.0, 0.0)
    offs = jnp.dot(lower, incl[:, 127:128], precision=_HIGH)

    rank = (incl + offs - kf).astype(jnp.int32)
    slot = jnp.where(keep, rank, jnp.int32(2**30))
    slot_flat = slot.reshape(1, _NPAD)
    p_i = jax.lax.broadcasted_iota(jnp.int32, (_SLOTS, 1), 0)
    onehot = jnp.where(p_i == slot_flat, 1.0, 0.0)
    res = jnp.dot(onehot, data_ref[...], precision=_HIGH)
    o_ref[...] = res[:_K, :5]


@functools.partial(jax.jit, static_argnames=("interpret",))
def kernel(boxes, scores, interpret=False):
    neg, x1c, y1c, x2c, y2c = jax.lax.sort(
        (-scores, boxes[:, 0], boxes[:, 1], boxes[:, 2], boxes[:, 3]),
        num_keys=1, is_stable=True)
    z = jnp.zeros((_N,), jnp.float32)
    data = jnp.stack([-neg, x1c, y1c, x2c, y2c, z, z, z], axis=1)
    data = jnp.pad(data, ((0, _NPAD - _N), (0, 0)))

    dmax = pl.pallas_call(
        _supp_kernel,
        grid_spec=pltpu.PrefetchScalarGridSpec(
            num_scalar_prefetch=1,
            grid=(len(_STEPS),),
            in_specs=[
                pl.BlockSpec((_R, 8), lambda i, tb: (tb[1, i], 0)),
                pl.BlockSpec((_C, 8), lambda i, tb: (tb[0, i], 0)),
            ],
            out_specs=pl.BlockSpec((1, _CR, 128),
                                   lambda i, tb: (tb[0, i], 0, 0)),
        ),
        out_shape=jax.ShapeDtypeStruct((_NB, _CR, 128), jnp.float32),
        compiler_params=pltpu.CompilerParams(
            dimension_semantics=("parallel",)),
        interpret=interpret,
    )(jnp.asarray(_TABLE), data, data)

    return pl.pallas_call(
        _compact_kernel,
        out_shape=jax.ShapeDtypeStruct((_K, 5), jnp.float32),
        interpret=interpret,
    )(dmax, data)
